# baseline (speedup 1.0000x reference)
import functools

import jax
import jax.numpy as jnp
from jax.experimental import pallas as pl


_B, _C2, _N = 16, 128, 1024
_C = _C2 // 2
_Q, _K, _D = 4, 1024, 64

@functools.lru_cache(maxsize=1)
def _noise_const():
    import numpy as np
    try:
        with jax.ensure_compile_time_eval():
            return np.asarray(
                jax.random.normal(jax.random.key(42), (_B, _C, _N),
                                  dtype=jnp.float32))
    except Exception:
        return None


def _rvq_kernel(x_ref, noise_ref, cb_ref, o_ref):
    nbat = x_ref.shape[0]
    rs = []
    for b in range(nbat):
        xb = x_ref[b]
        mean = xb[:_C, :]
        scale = xb[_C:, :]
        sp = jnp.maximum(scale, 0.0) + jnp.log1p(jnp.exp(-jnp.abs(scale)))
        stdev = sp + 0.0001
        rs.append(noise_ref[b] * stdev + mean)
    ngrp = 2 if nbat >= 2 else 1
    per = nbat // ngrp
    groups = [
        jnp.concatenate(rs[g * per:(g + 1) * per], axis=1) if per > 1
        else rs[g * per]
        for g in range(ngrp)
    ]
    nb = groups[0].shape[1]
    row_iota = jax.lax.broadcasted_iota(jnp.int32, (_K, nb), 0).astype(jnp.float32)

    preps = []
    for q in range(_Q):
        cb = cb_ref[q]
        c2 = jnp.sum(cb * cb, axis=1)[:, None]
        cbm2 = -2.0 * cb
        c_hi16 = cb.astype(jnp.bfloat16)
        rem1 = cb - c_hi16.astype(jnp.float32)
        c_mid16 = rem1.astype(jnp.bfloat16)
        c_lo16 = (rem1 - c_mid16.astype(jnp.float32)).astype(jnp.bfloat16)
        c_stack = jnp.concatenate([c_hi16, c_mid16, c_lo16], axis=1)
        preps.append((c2, cbm2, c_stack))

    accs = [jnp.zeros((_C, nb), dtype=jnp.float32) for _ in range(ngrp)]
    for q in range(_Q):
        c2, cbm2, c_stack = preps[q]
        for g in range(ngrp):
            r = groups[g]
            r2 = jnp.sum(r * r, axis=0, keepdims=True)
            scores = jax.lax.dot_general(
                cbm2, r, (((1,), (0,)), ((), ())),
                preferred_element_type=jnp.float32)
            d = (r2 + scores) + c2
            m = jnp.min(d, axis=0, keepdims=True)
            w = jnp.where(d == m, row_iota, float(_K))
            idx = jnp.min(w, axis=0, keepdims=True)
            onehot = (row_iota == idx).astype(jnp.float32).astype(jnp.bfloat16)
            parts = jax.lax.dot_general(
                c_stack, onehot, (((0,), (0,)), ((), ())),
                preferred_element_type=jnp.float32)
            quant = (parts[0:_C] + parts[_C:2 * _C]) + parts[2 * _C:3 * _C]
            accs[g] = accs[g] + quant
            groups[g] = r - quant

    acc = jnp.concatenate(accs, axis=1) if ngrp > 1 else accs[0]
    for b in range(nbat):
        o_ref[b] = acc[:, b * _N:(b + 1) * _N]


@functools.partial(jax.jit, static_argnames=())
def kernel(x, codebooks):
    cached = _noise_const()
    if cached is not None:
        noise = jnp.asarray(cached)
    else:
        noise = jax.random.normal(jax.random.key(42), (_B, _C, _N),
                                  dtype=jnp.float32)
    nbat = 4
    grid = (_B // nbat,)
    return pl.pallas_call(
        _rvq_kernel,
        grid=grid,
        in_specs=[
            pl.BlockSpec((nbat, _C2, _N), lambda b: (b, 0, 0)),
            pl.BlockSpec((nbat, _C, _N), lambda b: (b, 0, 0)),
            pl.BlockSpec((_Q, _K, _D), lambda b: (0, 0, 0)),
        ],
        out_specs=pl.BlockSpec((nbat, _C, _N), lambda b: (b, 0, 0)),
        out_shape=jax.ShapeDtypeStruct((_B, _C, _N), jnp.float32),
    )(x, noise, codebooks)


# speedup vs baseline: 1.0158x; 1.0158x over previous
import functools

import jax
import jax.numpy as jnp
from jax.experimental import pallas as pl


_B, _C2, _N = 16, 128, 1024
_C = _C2 // 2
_Q, _K, _D = 4, 1024, 64

@functools.lru_cache(maxsize=1)
def _noise_const():
    import numpy as np
    try:
        with jax.ensure_compile_time_eval():
            return np.asarray(
                jax.random.normal(jax.random.key(42), (_B, _C, _N),
                                  dtype=jnp.float32))
    except Exception:
        return None


def _rvq_kernel(x_ref, noise_ref, cb_ref, o_ref):
    nbat = x_ref.shape[0]
    rs = []
    for b in range(nbat):
        xb = x_ref[b]
        mean = xb[:_C, :]
        scale = xb[_C:, :]
        sp = jnp.maximum(scale, 0.0) + jnp.log1p(jnp.exp(-jnp.abs(scale)))
        stdev = sp + 0.0001
        rs.append(noise_ref[b] * stdev + mean)
    ngrp = 1
    per = nbat // ngrp
    groups = [
        jnp.concatenate(rs[g * per:(g + 1) * per], axis=1) if per > 1
        else rs[g * per]
        for g in range(ngrp)
    ]
    nb = groups[0].shape[1]
    row_iota = jax.lax.broadcasted_iota(jnp.int32, (_K, nb), 0).astype(jnp.float32)

    preps = []
    for q in range(_Q):
        cb = cb_ref[q]
        c2 = jnp.sum(cb * cb, axis=1)[:, None]
        cbm2 = -2.0 * cb
        c_hi16 = cb.astype(jnp.bfloat16)
        rem1 = cb - c_hi16.astype(jnp.float32)
        c_mid16 = rem1.astype(jnp.bfloat16)
        c_lo16 = (rem1 - c_mid16.astype(jnp.float32)).astype(jnp.bfloat16)
        c_stack = jnp.concatenate([c_hi16, c_mid16, c_lo16], axis=1)
        preps.append((c2, cbm2, c_stack))

    accs = [jnp.zeros((_C, nb), dtype=jnp.float32) for _ in range(ngrp)]
    for q in range(_Q):
        c2, cbm2, c_stack = preps[q]
        for g in range(ngrp):
            r = groups[g]
            r2 = jnp.sum(r * r, axis=0, keepdims=True)
            scores = jax.lax.dot_general(
                cbm2, r, (((1,), (0,)), ((), ())),
                preferred_element_type=jnp.float32)
            d = (r2 + scores) + c2
            m = jnp.min(d, axis=0, keepdims=True)
            w = jnp.where(d == m, row_iota, float(_K))
            idx = jnp.min(w, axis=0, keepdims=True)
            onehot = (row_iota == idx).astype(jnp.float32).astype(jnp.bfloat16)
            parts = jax.lax.dot_general(
                c_stack, onehot, (((0,), (0,)), ((), ())),
                preferred_element_type=jnp.float32)
            quant = (parts[0:_C] + parts[_C:2 * _C]) + parts[2 * _C:3 * _C]
            accs[g] = accs[g] + quant
            groups[g] = r - quant

    acc = jnp.concatenate(accs, axis=1) if ngrp > 1 else accs[0]
    for b in range(nbat):
        o_ref[b] = acc[:, b * _N:(b + 1) * _N]


@functools.partial(jax.jit, static_argnames=())
def kernel(x, codebooks):
    cached = _noise_const()
    if cached is not None:
        noise = jnp.asarray(cached)
    else:
        noise = jax.random.normal(jax.random.key(42), (_B, _C, _N),
                                  dtype=jnp.float32)
    nbat = 4
    grid = (_B // nbat,)
    return pl.pallas_call(
        _rvq_kernel,
        grid=grid,
        in_specs=[
            pl.BlockSpec((nbat, _C2, _N), lambda b: (b, 0, 0)),
            pl.BlockSpec((nbat, _C, _N), lambda b: (b, 0, 0)),
            pl.BlockSpec((_Q, _K, _D), lambda b: (0, 0, 0)),
        ],
        out_specs=pl.BlockSpec((nbat, _C, _N), lambda b: (b, 0, 0)),
        out_shape=jax.ShapeDtypeStruct((_B, _C, _N), jnp.float32),
    )(x, noise, codebooks)


# speedup vs baseline: 1.0167x; 1.0008x over previous
import functools

import jax
import jax.numpy as jnp
from jax.experimental import pallas as pl


_B, _C2, _N = 16, 128, 1024
_C = _C2 // 2
_Q, _K, _D = 4, 1024, 64

@functools.lru_cache(maxsize=1)
def _noise_const():
    import numpy as np
    try:
        with jax.ensure_compile_time_eval():
            return np.asarray(
                jax.random.normal(jax.random.key(42), (_B, _C, _N),
                                  dtype=jnp.float32))
    except Exception:
        return None


def _rvq_kernel(x_ref, noise_ref, cb_ref, o_ref):
    nbat = x_ref.shape[0]
    rs = []
    for b in range(nbat):
        xb = x_ref[b]
        mean = xb[:_C, :]
        scale = xb[_C:, :]
        sp = jnp.maximum(scale, 0.0) + jnp.log1p(jnp.exp(-jnp.abs(scale)))
        stdev = sp + 0.0001
        rs.append(noise_ref[b] * stdev + mean)
    ngrp = 1
    per = nbat // ngrp
    groups = [
        jnp.concatenate(rs[g * per:(g + 1) * per], axis=1) if per > 1
        else rs[g * per]
        for g in range(ngrp)
    ]
    nb = groups[0].shape[1]
    row_iota = jax.lax.broadcasted_iota(jnp.int32, (_K, 1), 0).astype(jnp.float32)

    preps = []
    for q in range(_Q):
        cb = cb_ref[q]
        c2 = jnp.sum(cb * cb, axis=1)[:, None]
        cbm2 = -2.0 * cb
        c_hi16 = cb.astype(jnp.bfloat16)
        rem1 = cb - c_hi16.astype(jnp.float32)
        c_mid16 = rem1.astype(jnp.bfloat16)
        c_lo16 = (rem1 - c_mid16.astype(jnp.float32)).astype(jnp.bfloat16)
        c_stack = jnp.concatenate([c_hi16, c_mid16, c_lo16], axis=1)
        preps.append((c2, cbm2, c_stack))

    accs = [jnp.zeros((_C, nb), dtype=jnp.float32) for _ in range(ngrp)]
    for q in range(_Q):
        c2, cbm2, c_stack = preps[q]
        for g in range(ngrp):
            r = groups[g]
            r2 = jnp.sum(r * r, axis=0, keepdims=True)
            scores = jax.lax.dot_general(
                cbm2, r, (((1,), (0,)), ((), ())),
                preferred_element_type=jnp.float32)
            d = (r2 + scores) + c2
            m = jnp.min(d, axis=0, keepdims=True)
            w = jnp.where(d == m, row_iota, float(_K))
            idx = jnp.min(w, axis=0, keepdims=True)
            onehot = (row_iota == idx).astype(jnp.float32).astype(jnp.bfloat16)
            parts = jax.lax.dot_general(
                c_stack, onehot, (((0,), (0,)), ((), ())),
                preferred_element_type=jnp.float32)
            quant = (parts[0:_C] + parts[_C:2 * _C]) + parts[2 * _C:3 * _C]
            accs[g] = accs[g] + quant
            groups[g] = r - quant

    acc = jnp.concatenate(accs, axis=1) if ngrp > 1 else accs[0]
    for b in range(nbat):
        o_ref[b] = acc[:, b * _N:(b + 1) * _N]


@functools.partial(jax.jit, static_argnames=())
def kernel(x, codebooks):
    cached = _noise_const()
    if cached is not None:
        noise = jnp.asarray(cached)
    else:
        noise = jax.random.normal(jax.random.key(42), (_B, _C, _N),
                                  dtype=jnp.float32)
    nbat = 4
    grid = (_B // nbat,)
    return pl.pallas_call(
        _rvq_kernel,
        grid=grid,
        in_specs=[
            pl.BlockSpec((nbat, _C2, _N), lambda b: (b, 0, 0)),
            pl.BlockSpec((nbat, _C, _N), lambda b: (b, 0, 0)),
            pl.BlockSpec((_Q, _K, _D), lambda b: (0, 0, 0)),
        ],
        out_specs=pl.BlockSpec((nbat, _C, _N), lambda b: (b, 0, 0)),
        out_shape=jax.ShapeDtypeStruct((_B, _C, _N), jnp.float32),
    )(x, noise, codebooks)


# speedup vs baseline: 1.1908x; 1.1713x over previous
import functools

import jax
import jax.numpy as jnp
from jax.experimental import pallas as pl


_B, _C2, _N = 16, 128, 1024
_C = _C2 // 2
_Q, _K, _D = 4, 1024, 64

@functools.lru_cache(maxsize=1)
def _noise_const():
    import numpy as np
    try:
        with jax.ensure_compile_time_eval():
            return np.asarray(
                jax.random.normal(jax.random.key(42), (_B, _C, _N),
                                  dtype=jnp.float32))
    except Exception:
        return None


def _rvq_kernel(x_ref, noise_ref, cb_ref, o_ref):
    nbat = x_ref.shape[0]
    rs = []
    for b in range(nbat):
        xb = x_ref[b]
        mean = xb[:_C, :]
        scale = xb[_C:, :]
        sp = jnp.maximum(scale, 0.0) + jnp.log1p(jnp.exp(-jnp.abs(scale)))
        stdev = sp + 0.0001
        rs.append(noise_ref[b] * stdev + mean)
    ngrp = 1
    per = nbat // ngrp
    groups = [
        jnp.concatenate(rs[g * per:(g + 1) * per], axis=1) if per > 1
        else rs[g * per]
        for g in range(ngrp)
    ]
    nb = groups[0].shape[1]
    row_iota = jax.lax.broadcasted_iota(jnp.int32, (_K, 1), 0).astype(jnp.float32)

    preps = []
    for q in range(_Q):
        cb = cb_ref[q]
        c2 = jnp.sum(cb * cb, axis=1)[:, None]
        cbm2 = -2.0 * cb
        c_hi16 = cb.astype(jnp.bfloat16)
        rem1 = cb - c_hi16.astype(jnp.float32)
        c_mid16 = rem1.astype(jnp.bfloat16)
        c_lo16 = (rem1 - c_mid16.astype(jnp.float32)).astype(jnp.bfloat16)
        ones_col = jnp.ones((_K, 1), dtype=jnp.bfloat16)
        c_stack = jnp.concatenate([c_hi16, c_mid16, c_lo16, ones_col], axis=1)
        preps.append((c2, cbm2, c_stack))

    accs = [jnp.zeros((_C, nb), dtype=jnp.float32) for _ in range(ngrp)]
    for q in range(_Q):
        c2, cbm2, c_stack = preps[q]
        for g in range(ngrp):
            r = groups[g]
            r2 = jnp.sum(r * r, axis=0, keepdims=True)
            scores = jax.lax.dot_general(
                cbm2, r, (((1,), (0,)), ((), ())),
                preferred_element_type=jnp.float32)
            d = (r2 + scores) + c2
            m = jnp.min(d, axis=0, keepdims=True)
            mh = d == m
            hot = mh.astype(jnp.float32).astype(jnp.bfloat16)
            parts = jax.lax.dot_general(
                c_stack, hot, (((0,), (0,)), ((), ())),
                preferred_element_type=jnp.float32)
            cnt = parts[3 * _C:3 * _C + 1]
            has_tie = jnp.max(cnt) > 1.5

            def _fix(_):
                w = jnp.where(mh, row_iota, float(_K))
                idx = jnp.min(w, axis=0, keepdims=True)
                oh = (row_iota == idx).astype(jnp.float32).astype(jnp.bfloat16)
                p2 = jax.lax.dot_general(
                    c_stack, oh, (((0,), (0,)), ((), ())),
                    preferred_element_type=jnp.float32)
                return (p2[0:_C] + p2[_C:2 * _C]) + p2[2 * _C:3 * _C]

            def _fast(_):
                return (parts[0:_C] + parts[_C:2 * _C]) + parts[2 * _C:3 * _C]

            quant = jax.lax.cond(has_tie, _fix, _fast, None)
            accs[g] = accs[g] + quant
            groups[g] = r - quant

    acc = jnp.concatenate(accs, axis=1) if ngrp > 1 else accs[0]
    for b in range(nbat):
        o_ref[b] = acc[:, b * _N:(b + 1) * _N]


@functools.partial(jax.jit, static_argnames=())
def kernel(x, codebooks):
    cached = _noise_const()
    if cached is not None:
        noise = jnp.asarray(cached)
    else:
        noise = jax.random.normal(jax.random.key(42), (_B, _C, _N),
                                  dtype=jnp.float32)
    nbat = 4
    grid = (_B // nbat,)
    return pl.pallas_call(
        _rvq_kernel,
        grid=grid,
        in_specs=[
            pl.BlockSpec((nbat, _C2, _N), lambda b: (b, 0, 0)),
            pl.BlockSpec((nbat, _C, _N), lambda b: (b, 0, 0)),
            pl.BlockSpec((_Q, _K, _D), lambda b: (0, 0, 0)),
        ],
        out_specs=pl.BlockSpec((nbat, _C, _N), lambda b: (b, 0, 0)),
        out_shape=jax.ShapeDtypeStruct((_B, _C, _N), jnp.float32),
    )(x, noise, codebooks)
